# baseline (speedup 1.0000x reference)
import functools

import jax
import jax.numpy as jnp
from jax import lax
from jax.experimental import pallas as pl
from jax.experimental.pallas import tpu as pltpu
from jax.experimental.pallas import tpu_sc as plsc

N_IN = 16777216
N_OUT = 2 * N_IN
NUM_CORES = 2
NUM_SUBCORES = 16
NW = NUM_CORES * NUM_SUBCORES
S = N_IN // NW
C = 32768
NCHUNK = S // C

_mesh = plsc.VectorSubcoreMesh(
    core_axis_name="c", subcore_axis_name="s", num_cores=NUM_CORES)


@functools.partial(
    pl.kernel,
    mesh=_mesh,
    out_type=jax.ShapeDtypeStruct((N_OUT,), jnp.float32),
    scratch_types=[
        pltpu.VMEM((C,), jnp.float32),
        pltpu.SemaphoreType.DMA,
        pltpu.SemaphoreType.DMA,
    ],
)
def _add_ancilla(psi_hbm, out_hbm, zbuf, sem_c, sem_z):
    wid = lax.axis_index("s") * NUM_CORES + lax.axis_index("c")
    base = wid * S

    zero16 = jnp.zeros((16,), jnp.float32)

    def zero_body(i, _):
        for u in range(8):
            zbuf[pl.ds((i * 8 + u) * 16, 16)] = zero16
        return 0

    lax.fori_loop(0, C // (16 * 8), zero_body, 0)

    copies = []
    for i in range(NCHUNK):
        off = base + i * C
        copies.append(
            pltpu.async_copy(psi_hbm.at[pl.ds(off, C)],
                             out_hbm.at[pl.ds(off, C)], sem_c))
        copies.append(
            pltpu.async_copy(zbuf, out_hbm.at[pl.ds(N_IN + off, C)], sem_z))
    for cp in copies:
        cp.wait()


def kernel(psi):
    return _add_ancilla(psi)


# speedup vs baseline: 21.3012x; 21.3012x over previous
import functools

import jax
import jax.numpy as jnp
from jax import lax
from jax.experimental import pallas as pl
from jax.experimental.pallas import tpu as pltpu
from jax.experimental.pallas import tpu_sc as plsc

N_IN = 16777216
N_OUT = 2 * N_IN
NUM_CORES = 2
NUM_SUBCORES = 16
NW = NUM_CORES * NUM_SUBCORES
S = N_IN // NW
C = 32768
NCHUNK = S // C

_mesh = plsc.VectorSubcoreMesh(
    core_axis_name="c", subcore_axis_name="s", num_cores=NUM_CORES)


@functools.partial(
    pl.kernel,
    mesh=_mesh,
    out_type=jax.ShapeDtypeStruct((N_OUT,), jnp.float32),
    scratch_types=[
        pltpu.VMEM((C,), jnp.float32),
        pltpu.VMEM((C,), jnp.float32),
        pltpu.SemaphoreType.DMA,
    ],
)
def _add_ancilla(psi_hbm, out_hbm, buf, zbuf, sem_z):
    wid = lax.axis_index("s") * NUM_CORES + lax.axis_index("c")
    base = wid * S

    zero16 = jnp.zeros((16,), jnp.float32)

    def zero_body(i, _):
        for u in range(8):
            zbuf[pl.ds((i * 8 + u) * 16, 16)] = zero16
        return 0

    lax.fori_loop(0, C // (16 * 8), zero_body, 0)

    zcopies = [
        pltpu.async_copy(zbuf, out_hbm.at[pl.ds(N_IN + base + i * C, C)],
                         sem_z)
        for i in range(NCHUNK)
    ]

    def body(i, _):
        off = base + i * C
        pltpu.sync_copy(psi_hbm.at[pl.ds(off, C)], buf)
        pltpu.sync_copy(buf, out_hbm.at[pl.ds(off, C)])
        return 0

    lax.fori_loop(0, NCHUNK, body, 0)

    for cp in zcopies:
        cp.wait()


def kernel(psi):
    return _add_ancilla(psi)


# speedup vs baseline: 22.2532x; 1.0447x over previous
import functools

import jax
import jax.numpy as jnp
from jax import lax
from jax.experimental import pallas as pl
from jax.experimental.pallas import tpu as pltpu
from jax.experimental.pallas import tpu_sc as plsc

N_IN = 16777216
N_OUT = 2 * N_IN
NUM_CORES = 2
NUM_SUBCORES = 16
NW = NUM_CORES * NUM_SUBCORES
S = N_IN // NW
C = 32768
NCHUNK = S // C

_mesh = plsc.VectorSubcoreMesh(
    core_axis_name="c", subcore_axis_name="s", num_cores=NUM_CORES)


@functools.partial(
    pl.kernel,
    mesh=_mesh,
    out_type=jax.ShapeDtypeStruct((N_OUT,), jnp.float32),
    scratch_types=[
        pltpu.VMEM((C,), jnp.float32),
        pltpu.VMEM((C,), jnp.float32),
        pltpu.VMEM((C,), jnp.float32),
        pltpu.SemaphoreType.DMA,
        pltpu.SemaphoreType.DMA,
        pltpu.SemaphoreType.DMA,
        pltpu.SemaphoreType.DMA,
        pltpu.SemaphoreType.DMA,
    ],
)
def _add_ancilla(psi_hbm, out_hbm, buf_a, buf_b, zbuf,
                 sem_ra, sem_rb, sem_wa, sem_wb, sem_z):
    wid = lax.axis_index("s") * NUM_CORES + lax.axis_index("c")
    base = wid * S

    zero16 = jnp.zeros((16,), jnp.float32)

    def zero_body(i, _):
        for u in range(8):
            zbuf[pl.ds((i * 8 + u) * 16, 16)] = zero16
        return 0

    lax.fori_loop(0, C // (16 * 8), zero_body, 0)

    zcopies = [
        pltpu.async_copy(zbuf, out_hbm.at[pl.ds(N_IN + base + i * C, C)],
                         sem_z)
        for i in range(NCHUNK)
    ]

    bufs = (buf_a, buf_b)
    sem_r = (sem_ra, sem_rb)
    sem_w = (sem_wa, sem_wb)
    rd, wr = {}, {}
    for i in range(NCHUNK + 1):
        if i < NCHUNK:
            b = i % 2
            if i >= 2:
                wr[i - 2].wait()
            rd[i] = pltpu.async_copy(
                psi_hbm.at[pl.ds(base + i * C, C)], bufs[b], sem_r[b])
        if i >= 1:
            j = i - 1
            rd[j].wait()
            wr[j] = pltpu.async_copy(
                bufs[j % 2], out_hbm.at[pl.ds(base + j * C, C)],
                sem_w[j % 2])
    wr[NCHUNK - 2].wait()
    wr[NCHUNK - 1].wait()

    for cp in zcopies:
        cp.wait()


def kernel(psi):
    return _add_ancilla(psi)


# speedup vs baseline: 23.4072x; 1.0519x over previous
import functools

import jax
import jax.numpy as jnp
from jax import lax
from jax.experimental import pallas as pl
from jax.experimental.pallas import tpu as pltpu
from jax.experimental.pallas import tpu_sc as plsc

N_IN = 16777216
N_OUT = 2 * N_IN
NUM_CORES = 2
NUM_SUBCORES = 16
NW = NUM_CORES * NUM_SUBCORES
S = N_IN // NW
C = 32768
NCHUNK = S // C
NB = 3
Z = 16384
NZ = S // Z

_mesh = plsc.VectorSubcoreMesh(
    core_axis_name="c", subcore_axis_name="s", num_cores=NUM_CORES)


@functools.partial(
    pl.kernel,
    mesh=_mesh,
    out_type=jax.ShapeDtypeStruct((N_OUT,), jnp.float32),
    scratch_types=[
        pltpu.VMEM((C,), jnp.float32),
        pltpu.VMEM((C,), jnp.float32),
        pltpu.VMEM((C,), jnp.float32),
        pltpu.VMEM((Z,), jnp.float32),
        pltpu.SemaphoreType.DMA,
        pltpu.SemaphoreType.DMA,
        pltpu.SemaphoreType.DMA,
        pltpu.SemaphoreType.DMA,
        pltpu.SemaphoreType.DMA,
        pltpu.SemaphoreType.DMA,
        pltpu.SemaphoreType.DMA,
    ],
)
def _add_ancilla(psi_hbm, out_hbm, buf0, buf1, buf2, zbuf,
                 sem_r0, sem_r1, sem_r2, sem_w0, sem_w1, sem_w2, sem_z):
    wid = lax.axis_index("s") * NUM_CORES + lax.axis_index("c")
    base = wid * S
    bufs = (buf0, buf1, buf2)
    sem_r = (sem_r0, sem_r1, sem_r2)
    sem_w = (sem_w0, sem_w1, sem_w2)

    def read(i):
        b = i % NB
        return pltpu.async_copy(
            psi_hbm.at[pl.ds(base + i * C, C)], bufs[b], sem_r[b])

    def write(i):
        b = i % NB
        return pltpu.async_copy(
            bufs[b], out_hbm.at[pl.ds(base + i * C, C)], sem_w[b])

    rd, wr = {}, {}

    for i in range(NB):
        rd[i] = read(i)

    zero16 = jnp.zeros((16,), jnp.float32)

    def zero_body(i, _):
        for u in range(8):
            zbuf[pl.ds((i * 8 + u) * 16, 16)] = zero16
        return 0

    lax.fori_loop(0, Z // (16 * 8), zero_body, 0)

    zcopies = [
        pltpu.async_copy(zbuf, out_hbm.at[pl.ds(N_IN + base + i * Z, Z)],
                         sem_z)
        for i in range(NZ)
    ]

    for j in range(NB - 1):
        rd[j].wait()
        wr[j] = write(j)
    for i in range(NB, NCHUNK + 1):
        if i < NCHUNK:
            wr[i - NB].wait()
            rd[i] = read(i)
        j = i - 1
        rd[j].wait()
        wr[j] = write(j)
    for j in range(NCHUNK - NB, NCHUNK):
        wr[j].wait()

    for cp in zcopies:
        cp.wait()


def kernel(psi):
    return _add_ancilla(psi)


# speedup vs baseline: 23.5648x; 1.0067x over previous
import functools

import jax
import jax.numpy as jnp
from jax import lax
from jax.experimental import pallas as pl
from jax.experimental.pallas import tpu as pltpu
from jax.experimental.pallas import tpu_sc as plsc

N_IN = 16777216
N_OUT = 2 * N_IN
NUM_CORES = 2
NUM_SUBCORES = 16
NW = NUM_CORES * NUM_SUBCORES
S = N_IN // NW
C = 32768
NCHUNK = S // C
NB = 3

_mesh = plsc.VectorSubcoreMesh(
    core_axis_name="c", subcore_axis_name="s", num_cores=NUM_CORES)


@functools.partial(
    pl.kernel,
    mesh=_mesh,
    out_type=jax.ShapeDtypeStruct((N_OUT,), jnp.float32),
    scratch_types=[
        pltpu.VMEM((C,), jnp.float32),
        pltpu.VMEM((C,), jnp.float32),
        pltpu.VMEM((C,), jnp.float32),
        pltpu.SemaphoreType.DMA,
        pltpu.SemaphoreType.DMA,
        pltpu.SemaphoreType.DMA,
        pltpu.SemaphoreType.DMA,
        pltpu.SemaphoreType.DMA,
        pltpu.SemaphoreType.DMA,
    ],
)
def _copy_lower(psi_hbm, out_hbm, buf0, buf1, buf2,
                sem_r0, sem_r1, sem_r2, sem_w0, sem_w1, sem_w2):
    wid = lax.axis_index("s") * NUM_CORES + lax.axis_index("c")
    base = wid * S
    bufs = (buf0, buf1, buf2)
    sem_r = (sem_r0, sem_r1, sem_r2)
    sem_w = (sem_w0, sem_w1, sem_w2)

    def read(i):
        b = i % NB
        return pltpu.async_copy(
            psi_hbm.at[pl.ds(base + i * C, C)], bufs[b], sem_r[b])

    def write(i):
        b = i % NB
        return pltpu.async_copy(
            bufs[b], out_hbm.at[pl.ds(base + i * C, C)], sem_w[b])

    rd, wr = {}, {}
    for i in range(NB):
        rd[i] = read(i)
    for j in range(NB - 1):
        rd[j].wait()
        wr[j] = write(j)
    for i in range(NB, NCHUNK + 1):
        if i < NCHUNK:
            wr[i - NB].wait()
            rd[i] = read(i)
        j = i - 1
        rd[j].wait()
        wr[j] = write(j)
    for j in range(NCHUNK - NB, NCHUNK):
        wr[j].wait()


ZBLK = 1048576
NZBLK = N_IN // ZBLK


def _zero_upper_body(full_ref, out_ref):
    out_ref[...] = jnp.zeros((ZBLK,), jnp.float32)


_zero_upper = pl.pallas_call(
    _zero_upper_body,
    grid=(NZBLK,),
    in_specs=[pl.BlockSpec(memory_space=pl.ANY)],
    out_specs=pl.BlockSpec((ZBLK,), lambda i: (NZBLK + i,)),
    out_shape=jax.ShapeDtypeStruct((N_OUT,), jnp.float32),
    input_output_aliases={0: 0},
)


def kernel(psi):
    return _zero_upper(_copy_lower(psi))
